# baseline (speedup 1.0000x reference)
import functools

import jax
import jax.numpy as jnp
from jax import lax
from jax.experimental import pallas as pl
from jax.experimental.pallas import tpu as pltpu
from jax.experimental.pallas import tpu_sc as plsc

_B = 1024
_CTX = 20
_EM = 16
_V = 100000
_TN = 4096
_L = 16
_VPAD = 100096


def _mean_pool_sc(idx_ctx_major, em_t):
    info = plsc.get_sparse_core_info()
    nhalf = 2
    bpw = _B // nhalf
    ngroup = bpw // _L

    mesh = plsc.VectorSubcoreMesh(core_axis_name="c", subcore_axis_name="s")

    @functools.partial(
        pl.kernel,
        mesh=mesh,
        compiler_params=pltpu.CompilerParams(
            needs_layout_passes=False, use_tc_tiling_on_sc=True),
        out_type=jax.ShapeDtypeStruct((_EM * _B,), jnp.float32),
        scratch_types=[
            pltpu.VMEM((1, _V), jnp.float32),
            pltpu.VMEM((_CTX * _B,), jnp.int32),
            pltpu.VMEM((bpw,), jnp.float32),
            pltpu.SemaphoreType.DMA,
        ],
    )
    def k(idx_hbm, table_hbm, out_hbm, row_v, idx_v, ht_v, sem):
        f = lax.axis_index("s")
        half = lax.axis_index("c")
        b0 = half * bpw
        row_cp = pltpu.async_copy(
            table_hbm.at[pl.ds(f, 1), pl.ds(0, _V)], row_v, sem)
        idx_cp = pltpu.async_copy(idx_hbm, idx_v, sem)
        row_cp.wait()
        idx_cp.wait()
        zeros = jnp.zeros((_L,), jnp.int32)

        @plsc.parallel_loop(0, ngroup, 1, unroll=4)
        def body(g):
            base = b0 + g * _L
            acc = jnp.zeros((_L,), jnp.float32)
            for j in range(_CTX):
                eidx = idx_v[pl.ds(j * _B + base, _L)]
                acc = acc + plsc.load_gather(row_v, [zeros, eidx])
            ht_v[pl.ds(g * _L, _L)] = acc * (1.0 / _CTX)
        pltpu.sync_copy(ht_v, out_hbm.at[pl.ds(f * _B + b0, bpw)])

    return k(idx_ctx_major, em_t).reshape(_EM, _B)


def _proj_body(w_ref, h_ref, o_ref):
    o_ref[...] = lax.dot_general(
        w_ref[...],
        h_ref[...],
        (((0,), (0,)), ((), ())),
        preferred_element_type=jnp.float32,
    )


def _proj_tc(w_aug, h_aug):
    ka = _EM + 1
    return pl.pallas_call(
        _proj_body,
        grid=(pl.cdiv(_V, _TN),),
        in_specs=[
            pl.BlockSpec((ka, _TN), lambda i: (0, i)),
            pl.BlockSpec((ka, _B), lambda i: (0, 0)),
        ],
        out_specs=pl.BlockSpec((_TN, _B), lambda i: (i, 0)),
        out_shape=jax.ShapeDtypeStruct((_V, _B), jnp.float32),
    )(w_aug, h_aug)


def kernel(in_tensor, em_table, W, b):
    idx_cm = in_tensor.astype(jnp.int32).T.reshape(_CTX * _B)
    h_t = _mean_pool_sc(idx_cm, em_table.T)
    w_aug = jnp.concatenate([W.T, b.reshape(1, _V)], axis=0)
    h_aug = jnp.concatenate(
        [h_t, jnp.ones((1, _B), jnp.float32)], axis=0)
    o_t = _proj_tc(w_aug, h_aug)
    return o_t.T


# speedup vs baseline: 1.0090x; 1.0090x over previous
import functools

import jax
import jax.numpy as jnp
from jax import lax
from jax.experimental import pallas as pl
from jax.experimental.pallas import tpu as pltpu
from jax.experimental.pallas import tpu_sc as plsc

_B = 1024
_CTX = 20
_EM = 16
_V = 100000
_TN = 2048
_L = 16
_VPAD = 100096


def _mean_pool_sc(idx_t, em_t):
    info = plsc.get_sparse_core_info()
    nhalf = 2
    bpw = _B // nhalf
    ngroup = bpw // _L

    mesh = plsc.VectorSubcoreMesh(core_axis_name="c", subcore_axis_name="s")

    @functools.partial(
        pl.kernel,
        mesh=mesh,
        compiler_params=pltpu.CompilerParams(
            needs_layout_passes=False, use_tc_tiling_on_sc=True),
        out_type=jax.ShapeDtypeStruct((_EM * _B,), jnp.float32),
        scratch_types=[
            pltpu.VMEM((1, _V), jnp.float32),
            pltpu.VMEM((_CTX, bpw), jnp.int32),
            pltpu.VMEM((bpw,), jnp.float32),
            pltpu.SemaphoreType.DMA,
        ],
    )
    def k(idx_hbm, table_hbm, out_hbm, row_v, idx_v, ht_v, sem):
        f = lax.axis_index("s")
        half = lax.axis_index("c")
        b0 = half * bpw
        row_cp = pltpu.async_copy(
            table_hbm.at[pl.ds(f, 1), pl.ds(0, _V)], row_v, sem)
        idx_cp = pltpu.async_copy(
            idx_hbm.at[pl.ds(0, _CTX), pl.ds(b0, bpw)], idx_v, sem)
        row_cp.wait()
        idx_cp.wait()
        zeros = jnp.zeros((_L,), jnp.int32)
        lanes = lax.iota(jnp.int32, _L)

        @plsc.parallel_loop(0, ngroup, 1, unroll=4)
        def body(g):
            cols = lanes + g * _L
            acc = jnp.zeros((_L,), jnp.float32)
            for j in range(_CTX):
                eidx = plsc.load_gather(
                    idx_v, [jnp.full((_L,), j, jnp.int32), cols])
                acc = acc + plsc.load_gather(row_v, [zeros, eidx])
            ht_v[pl.ds(g * _L, _L)] = acc * (1.0 / _CTX)
        pltpu.sync_copy(ht_v, out_hbm.at[pl.ds(f * _B + b0, bpw)])

    return k(idx_t, em_t).reshape(_EM, _B)


def _proj_body(w_ref, h_ref, o_ref):
    o_ref[...] = lax.dot_general(
        w_ref[...],
        h_ref[...],
        (((0,), (0,)), ((), ())),
        preferred_element_type=jnp.float32,
    )


def _proj_tc(w_aug, h_aug):
    ka = _EM + 1
    return pl.pallas_call(
        _proj_body,
        grid=(pl.cdiv(_V, _TN),),
        in_specs=[
            pl.BlockSpec((ka, _TN), lambda i: (0, i)),
            pl.BlockSpec((ka, _B), lambda i: (0, 0)),
        ],
        out_specs=pl.BlockSpec((_TN, _B), lambda i: (i, 0)),
        out_shape=jax.ShapeDtypeStruct((_V, _B), jnp.float32),
    )(w_aug, h_aug)


def kernel(in_tensor, em_table, W, b):
    h_t = _mean_pool_sc(in_tensor.astype(jnp.int32).T, em_table.T)
    w_aug = jnp.concatenate([W.T, b.reshape(1, _V)], axis=0)
    h_aug = jnp.concatenate(
        [h_t, jnp.ones((1, _B), jnp.float32)], axis=0)
    o_t = _proj_tc(w_aug, h_aug)
    return o_t.T
